# baseline (device time: 53309 ns/iter reference)
import jax
import jax.numpy as jnp
from jax import lax
from jax.experimental import pallas as pl
from jax.experimental.pallas import tpu as pltpu

N_Z = 4
B, SKV, H, D = 16, 1024, 16, 64
SCALE = D ** -0.5
CHUNK = 4
NC = B // CHUNK
BPG = 2
NG = B // BPG
SPC = CHUNK // BPG


def _body(q_ref, k_ref, v_ref, out_ref, own, comm, send_sems, recv_sems):
    b = pl.program_id(0)
    my_x = lax.axis_index("x")
    my_y = lax.axis_index("y")
    my_z = lax.axis_index("z")

    @pl.when(b == 0)
    def _():
        barrier_sem = pltpu.get_barrier_semaphore()
        for off in (1, 2, 3):
            pl.semaphore_signal(
                barrier_sem, inc=1,
                device_id=(my_x, my_y, (my_z + off) % N_Z),
                device_id_type=pl.DeviceIdType.MESH,
            )

    q = q_ref[...].reshape(BPG * H, D)
    k = k_ref[...].reshape(BPG * H, D, SKV)
    v = v_ref[...].reshape(BPG * H, D, SKV)
    s = lax.dot_general(
        q, k, (((1,), (1,)), ((0,), (0,))),
        preferred_element_type=jnp.float32,
    )
    p = jnp.exp(s * SCALE)
    o = lax.dot_general(
        p, v, (((1,), (2,)), ((0,), (0,))),
        preferred_element_type=jnp.float32,
    )
    l = jnp.sum(p, axis=1, keepdims=True)
    own[pl.ds(b * BPG, BPG)] = jnp.concatenate([o, l], axis=1).reshape(
        BPG, H, D + 1)

    def _mk(c, off):
        return pltpu.make_async_remote_copy(
            src_ref=own.at[pl.ds(c * CHUNK, CHUNK)],
            dst_ref=comm.at[3 - off, pl.ds(c * CHUNK, CHUNK)],
            send_sem=send_sems.at[c, off - 1],
            recv_sem=recv_sems.at[c, 3 - off],
            device_id=(my_x, my_y, (my_z + off) % N_Z),
            device_id_type=pl.DeviceIdType.MESH,
        )

    for c in range(NC):
        @pl.when(b == c * SPC + SPC - 1)
        def _(c=c):
            if c == 0:
                pl.semaphore_wait(pltpu.get_barrier_semaphore(), 3)
            for off in (3, 2, 1):
                _mk(c, off).start()

    @pl.when(b == NG - 1)
    def _():
        head = (NC - 1) * CHUNK
        for c in range(NC - 1):
            for off in (1, 2, 3):
                _mk(c, off).wait()
        tot = (own[:head] + comm[0, :head] + comm[1, :head]
               + comm[2, :head])
        out_ref[:head] = (tot[:, :, :D] / tot[:, :, D:])[:, None, :, :]
        for off in (1, 2, 3):
            _mk(NC - 1, off).wait()
        tot = (own[head:] + comm[0, head:] + comm[1, head:]
               + comm[2, head:])
        out_ref[head:] = (tot[:, :, :D] / tot[:, :, D:])[:, None, :, :]


def kernel(Q, K, V):
    Qs = Q.reshape(B, H, D)
    Kt = jnp.transpose(K, (0, 2, 3, 1))
    Vt = jnp.transpose(V, (0, 2, 3, 1))

    return pl.pallas_call(
        _body,
        grid=(NG,),
        in_specs=[
            pl.BlockSpec((BPG, H, D), lambda b: (b, 0, 0)),
            pl.BlockSpec((BPG, H, D, SKV), lambda b: (b, 0, 0, 0)),
            pl.BlockSpec((BPG, H, D, SKV), lambda b: (b, 0, 0, 0)),
        ],
        out_specs=pl.BlockSpec((B, 1, H, D), lambda b: (0, 0, 0, 0)),
        out_shape=jax.ShapeDtypeStruct((B, 1, H, D), jnp.float32),
        scratch_shapes=[
            pltpu.VMEM((B, H, D + 1), jnp.float32),
            pltpu.VMEM((3, B, H, D + 1), jnp.float32),
            pltpu.SemaphoreType.DMA((NC, 3)),
            pltpu.SemaphoreType.DMA((NC, 3)),
        ],
        compiler_params=pltpu.CompilerParams(
            collective_id=0, vmem_limit_bytes=64 * 1024 * 1024,
        ),
    )(Qs, Kt, Vt)


# device time: 48942 ns/iter; 1.0892x vs baseline; 1.0892x over previous
import jax
import jax.numpy as jnp
from jax import lax
from jax.experimental import pallas as pl
from jax.experimental.pallas import tpu as pltpu

N_Z = 4
B, SKV, H, D = 16, 1024, 16, 64
SCALE = D ** -0.5
CHUNK = 4
NC = B // CHUNK


def _body(q_ref, k_ref, v_ref, out_ref, own, comm, send_sems, recv_sems):
    b = pl.program_id(0)
    my_x = lax.axis_index("x")
    my_y = lax.axis_index("y")
    my_z = lax.axis_index("z")

    @pl.when(b == 0)
    def _():
        barrier_sem = pltpu.get_barrier_semaphore()
        for off in (1, 2, 3):
            pl.semaphore_signal(
                barrier_sem, inc=1,
                device_id=(my_x, my_y, (my_z + off) % N_Z),
                device_id_type=pl.DeviceIdType.MESH,
            )

    q = q_ref[0]
    k = k_ref[0]
    v = v_ref[0]
    s = lax.dot_general(
        q, k, (((1,), (1,)), ((0,), (0,))),
        preferred_element_type=jnp.float32,
    )
    p = jnp.exp(s * SCALE)
    o = lax.dot_general(
        p, v, (((1,), (2,)), ((0,), (0,))),
        preferred_element_type=jnp.float32,
    )
    l = jnp.sum(p, axis=1, keepdims=True)
    own[pl.ds(b, 1)] = jnp.concatenate([o, l], axis=1)[None]

    def _mk(c, off):
        return pltpu.make_async_remote_copy(
            src_ref=own.at[pl.ds(c * CHUNK, CHUNK)],
            dst_ref=comm.at[3 - off, pl.ds(c * CHUNK, CHUNK)],
            send_sem=send_sems.at[c, off - 1],
            recv_sem=recv_sems.at[c, 3 - off],
            device_id=(my_x, my_y, (my_z + off) % N_Z),
            device_id_type=pl.DeviceIdType.MESH,
        )

    for c in range(NC):
        @pl.when(b == c * CHUNK + CHUNK - 1)
        def _(c=c):
            if c == 0:
                pl.semaphore_wait(pltpu.get_barrier_semaphore(), 3)
            for off in (1, 2, 3):
                _mk(c, off).start()

    @pl.when(b == B - 1)
    def _():
        for c in range(NC):
            for off in (1, 2, 3):
                _mk(c, off).wait()
        tot = own[...] + comm[0] + comm[1] + comm[2]
        out_ref[...] = (tot[:, :, :D] / tot[:, :, D:])[:, None, :, :]


def kernel(Q, K, V):
    Qs = Q.reshape(B, H, D)
    Kt = jnp.transpose(K, (0, 2, 3, 1))
    Vt = jnp.transpose(V, (0, 2, 3, 1))

    return pl.pallas_call(
        _body,
        grid=(B,),
        in_specs=[
            pl.BlockSpec((1, H, D), lambda b: (b, 0, 0)),
            pl.BlockSpec((1, H, D, SKV), lambda b: (b, 0, 0, 0)),
            pl.BlockSpec((1, H, D, SKV), lambda b: (b, 0, 0, 0)),
        ],
        out_specs=pl.BlockSpec((B, 1, H, D), lambda b: (0, 0, 0, 0)),
        out_shape=jax.ShapeDtypeStruct((B, 1, H, D), jnp.float32),
        scratch_shapes=[
            pltpu.VMEM((B, H, D + 1), jnp.float32),
            pltpu.VMEM((3, B, H, D + 1), jnp.float32),
            pltpu.SemaphoreType.DMA((NC, 3)),
            pltpu.SemaphoreType.DMA((NC, 3)),
        ],
        compiler_params=pltpu.CompilerParams(collective_id=0),
    )(Qs, Kt, Vt)
